# Initial kernel scaffold; baseline (speedup 1.0000x reference)
import functools

import jax
import jax.numpy as jnp
from jax import lax
from jax.experimental import pallas as pl
from jax.experimental.pallas import tpu as pltpu
from jax.experimental.pallas import tpu_sc as plsc

_NC = 2
_NS = 16
_NW = _NC * _NS

_N = 10000
_E = 320000
_D = 128
_DE = 16
_H = 128

_IDXR = _E // 128
_BR = 4
_NB = _IDXR // _BR
_ITERS = -(-_NB // _NW)
_RPT = _N // _NS
_CH = 125
_BE = _BR * 128


def _sc_mesh():
    return plsc.VectorSubcoreMesh(
        core_axis_name="c", subcore_axis_name="s",
        num_cores=_NC, num_subcores=_NS)


def _deg_body(dst2d, ones_hbm, z16_hbm, degp, acc16, dst_v, ones_v, z16_v,
              sem):
    c = lax.axis_index("c")
    s = lax.axis_index("s")
    wid = c * _NS + s

    pltpu.sync_copy(z16_hbm, z16_v)
    pltpu.sync_copy(ones_hbm, ones_v)
    pltpu.sync_copy(z16_v, acc16.at[pl.ds(s * _RPT, _RPT)])
    plsc.subcore_barrier()

    def body(i, carry):
        bi = i * _NW + wid

        @pl.when(bi < _NB)
        def _():
            pltpu.sync_copy(dst2d.at[pl.ds(bi * _BR, _BR)], dst_v)
            for k in range(_BR):
                pltpu.sync_copy(ones_v, acc16.at[dst_v.at[k]], add=True)

        return carry

    lax.fori_loop(0, _ITERS, body, None)
    plsc.subcore_barrier()
    pltpu.sync_copy(acc16.at[pl.ds(s * _RPT, _RPT)], z16_v)
    pltpu.sync_copy(z16_v, degp.at[c, pl.ds(s * _RPT, _RPT)])


def _deg_call(dst2d, ones, z16):
    return pl.kernel(
        _deg_body,
        out_type=jax.ShapeDtypeStruct((_NC, _N, 16), jnp.float32),
        mesh=_sc_mesh(),
        scratch_types=[
            pltpu.VMEM_SHARED((_N, 16), jnp.float32),
            pltpu.VMEM((_BR, 128), jnp.int32),
            pltpu.VMEM((128, 16), jnp.float32),
            pltpu.VMEM((_RPT, 16), jnp.float32),
            pltpu.SemaphoreType.DMA,
        ],
    )(dst2d, ones, z16)


def _s0_body(hq_hbm, dis16_hbm, ea_hbm, src2d, dst2d, z128_hbm, z16_hbm,
             sp, ap, acc, acc16, src_v, dst_v, rows_v, disr_v, ea_v,
             z128_v, z16_v, semg, semd, seme):
    c = lax.axis_index("c")
    s = lax.axis_index("s")
    wid = c * _NS + s

    pltpu.sync_copy(z128_hbm, z128_v)
    pltpu.sync_copy(z16_hbm, z16_v)
    for k in range(_RPT // _CH):
        pltpu.sync_copy(z128_v, acc.at[pl.ds(s * _RPT + k * _CH, _CH)])
    pltpu.sync_copy(z16_v, acc16.at[pl.ds(s * _RPT, _RPT)])
    plsc.subcore_barrier()

    def body(i, carry):
        bi = i * _NW + wid

        @pl.when(bi < _NB)
        def _():
            pltpu.sync_copy(src2d.at[pl.ds(bi * _BR, _BR)], src_v)
            pltpu.sync_copy(dst2d.at[pl.ds(bi * _BR, _BR)], dst_v)
            gcp = [pltpu.async_copy(hq_hbm.at[src_v.at[k]],
                                    rows_v.at[pl.ds(k * 128, 128)], semg)
                   for k in range(_BR)]
            dcp = [pltpu.async_copy(dis16_hbm.at[src_v.at[k]],
                                    disr_v.at[pl.ds(k * 128, 128)], semd)
                   for k in range(_BR)]
            ecp = pltpu.async_copy(ea_hbm.at[pl.ds(bi * _BE, _BE)], ea_v,
                                   seme)
            for cp in dcp:
                cp.wait()
            ecp.wait()

            def mbody(j, mc):
                ea_v[j, :] = ea_v[j, :] * disr_v[j, :]
                return mc

            lax.fori_loop(0, _BE, mbody, None, unroll=8)
            for cp in gcp:
                cp.wait()
            for k in range(_BR):
                pltpu.sync_copy(rows_v.at[pl.ds(k * 128, 128)],
                                acc.at[dst_v.at[k]], add=True)
                pltpu.sync_copy(ea_v.at[pl.ds(k * 128, 128)],
                                acc16.at[dst_v.at[k]], add=True)

        return carry

    lax.fori_loop(0, _ITERS, body, None)
    plsc.subcore_barrier()
    for k in range(_RPT // _CH):
        pltpu.sync_copy(acc.at[pl.ds(s * _RPT + k * _CH, _CH)], z128_v)
        pltpu.sync_copy(z128_v, sp.at[c, pl.ds(s * _RPT + k * _CH, _CH)])
    pltpu.sync_copy(acc16.at[pl.ds(s * _RPT, _RPT)], z16_v)
    pltpu.sync_copy(z16_v, ap.at[c, pl.ds(s * _RPT, _RPT)])


def _s0_call(hq, dis16, ea, src2d, dst2d, z128, z16):
    return pl.kernel(
        _s0_body,
        out_type=(jax.ShapeDtypeStruct((_NC, _N, _H), jnp.float32),
                  jax.ShapeDtypeStruct((_NC, _N, 16), jnp.float32)),
        mesh=_sc_mesh(),
        scratch_types=[
            pltpu.VMEM_SHARED((_N, _H), jnp.float32),
            pltpu.VMEM_SHARED((_N, 16), jnp.float32),
            pltpu.VMEM((_BR, 128), jnp.int32),
            pltpu.VMEM((_BR, 128), jnp.int32),
            pltpu.VMEM((_BE, _H), jnp.float32),
            pltpu.VMEM((_BE, 16), jnp.float32),
            pltpu.VMEM((_BE, 16), jnp.float32),
            pltpu.VMEM((_CH, _H), jnp.float32),
            pltpu.VMEM((_RPT, 16), jnp.float32),
            pltpu.SemaphoreType.DMA,
            pltpu.SemaphoreType.DMA,
            pltpu.SemaphoreType.DMA,
        ],
    )(hq, dis16, ea, src2d, dst2d, z128, z16)


def _s1_body(hq_hbm, src2d, dst2d, z128_hbm, sp, acc, src_v, dst_v, rows_v,
             z128_v, semg):
    c = lax.axis_index("c")
    s = lax.axis_index("s")
    wid = c * _NS + s

    pltpu.sync_copy(z128_hbm, z128_v)
    for k in range(_RPT // _CH):
        pltpu.sync_copy(z128_v, acc.at[pl.ds(s * _RPT + k * _CH, _CH)])
    plsc.subcore_barrier()

    def body(i, carry):
        bi = i * _NW + wid

        @pl.when(bi < _NB)
        def _():
            pltpu.sync_copy(src2d.at[pl.ds(bi * _BR, _BR)], src_v)
            pltpu.sync_copy(dst2d.at[pl.ds(bi * _BR, _BR)], dst_v)
            gcp = [pltpu.async_copy(hq_hbm.at[src_v.at[k]],
                                    rows_v.at[pl.ds(k * 128, 128)], semg)
                   for k in range(_BR)]
            for cp in gcp:
                cp.wait()
            for k in range(_BR):
                pltpu.sync_copy(rows_v.at[pl.ds(k * 128, 128)],
                                acc.at[dst_v.at[k]], add=True)

        return carry

    lax.fori_loop(0, _ITERS, body, None)
    plsc.subcore_barrier()
    for k in range(_RPT // _CH):
        pltpu.sync_copy(acc.at[pl.ds(s * _RPT + k * _CH, _CH)], z128_v)
        pltpu.sync_copy(z128_v, sp.at[c, pl.ds(s * _RPT + k * _CH, _CH)])


def _s1_call(hq, src2d, dst2d, z128):
    return pl.kernel(
        _s1_body,
        out_type=jax.ShapeDtypeStruct((_NC, _N, _H), jnp.float32),
        mesh=_sc_mesh(),
        scratch_types=[
            pltpu.VMEM_SHARED((_N, _H), jnp.float32),
            pltpu.VMEM((_BR, 128), jnp.int32),
            pltpu.VMEM((_BR, 128), jnp.int32),
            pltpu.VMEM((_BE, _H), jnp.float32),
            pltpu.VMEM((_CH, _H), jnp.float32),
            pltpu.SemaphoreType.DMA,
        ],
    )(hq, src2d, dst2d, z128)


_TBR = 400
_TG = _N // _TBR


def _tc1_body(degp_ref, x_ref, wn_ref, b_ref, hq_ref, dis16_ref):
    deg16 = degp_ref[0] + degp_ref[1]
    dis16 = lax.rsqrt(deg16 + 1.0)
    hp = jnp.dot(x_ref[...], wn_ref[...],
                 preferred_element_type=jnp.float32) + b_ref[...]
    hq_ref[...] = hp * dis16[:, 0:1]
    dis16_ref[...] = dis16


def _tc1_call(degp, x, wn, b):
    return pl.pallas_call(
        _tc1_body,
        grid=(_TG,),
        in_specs=[
            pl.BlockSpec((_NC, _TBR, 16), lambda i: (0, i, 0)),
            pl.BlockSpec((_TBR, _D), lambda i: (i, 0)),
            pl.BlockSpec((_D, _H), lambda i: (0, 0)),
            pl.BlockSpec((1, _H), lambda i: (0, 0)),
        ],
        out_specs=[
            pl.BlockSpec((_TBR, _H), lambda i: (i, 0)),
            pl.BlockSpec((_TBR, 16), lambda i: (i, 0)),
        ],
        out_shape=[jax.ShapeDtypeStruct((_N, _H), jnp.float32),
                   jax.ShapeDtypeStruct((_N, 16), jnp.float32)],
    )(degp, x, wn, b)


def _tca_body(sp_ref, ap_ref, we_ref, dis16_ref, h_ref, a_ref, stats_ref):
    i = pl.program_id(0)
    a = ap_ref[0] + ap_ref[1]
    s = sp_ref[0] + sp_ref[1]
    h = (s + jnp.dot(a, we_ref[...], preferred_element_type=jnp.float32)
         ) * dis16_ref[:, 0:1]
    h_ref[...] = h
    a_ref[...] = a

    @pl.when(i == 0)
    def _():
        stats_ref[...] = jnp.zeros_like(stats_ref)

    stats_ref[0:1, :] += jnp.sum(h, axis=0, keepdims=True)
    stats_ref[1:2, :] += jnp.sum(h * h, axis=0, keepdims=True)


def _tca_call(sp, ap, we, dis16):
    return pl.pallas_call(
        _tca_body,
        grid=(_TG,),
        in_specs=[
            pl.BlockSpec((_NC, _TBR, _H), lambda i: (0, i, 0)),
            pl.BlockSpec((_NC, _TBR, 16), lambda i: (0, i, 0)),
            pl.BlockSpec((_DE, _H), lambda i: (0, 0)),
            pl.BlockSpec((_TBR, 16), lambda i: (i, 0)),
        ],
        out_specs=[
            pl.BlockSpec((_TBR, _H), lambda i: (i, 0)),
            pl.BlockSpec((_TBR, 16), lambda i: (i, 0)),
            pl.BlockSpec((8, _H), lambda i: (0, 0)),
        ],
        out_shape=[jax.ShapeDtypeStruct((_N, _H), jnp.float32),
                   jax.ShapeDtypeStruct((_N, 16), jnp.float32),
                   jax.ShapeDtypeStruct((8, _H), jnp.float32)],
    )(sp, ap, we, dis16)


def _tc2b_body(h_ref, stats_ref, g_ref, bb_ref, wn_ref, b_ref, dis16_ref,
               hq_ref):
    mean = stats_ref[0:1, :] * (1.0 / _N)
    var = stats_ref[1:2, :] * (1.0 / _N) - mean * mean
    sc = lax.rsqrt(var + 1e-5) * g_ref[...]
    bn = (h_ref[...] - mean) * sc + bb_ref[...]
    r = jnp.maximum(bn, 0.0)
    hq_ref[...] = (jnp.dot(r, wn_ref[...],
                           preferred_element_type=jnp.float32)
                   + b_ref[...]) * dis16_ref[:, 0:1]


def _tc2b_call(h, stats, g, bb, wn, b, dis16):
    return pl.pallas_call(
        _tc2b_body,
        grid=(_TG,),
        in_specs=[
            pl.BlockSpec((_TBR, _H), lambda i: (i, 0)),
            pl.BlockSpec((8, _H), lambda i: (0, 0)),
            pl.BlockSpec((1, _H), lambda i: (0, 0)),
            pl.BlockSpec((1, _H), lambda i: (0, 0)),
            pl.BlockSpec((_H, _H), lambda i: (0, 0)),
            pl.BlockSpec((1, _H), lambda i: (0, 0)),
            pl.BlockSpec((_TBR, 16), lambda i: (i, 0)),
        ],
        out_specs=pl.BlockSpec((_TBR, _H), lambda i: (i, 0)),
        out_shape=jax.ShapeDtypeStruct((_N, _H), jnp.float32),
    )(h, stats, g, bb, wn, b, dis16)


def _tc3a_body(sp_ref, a_ref, we_ref, dis16_ref, h_ref, stats_ref):
    i = pl.program_id(0)
    s = sp_ref[0] + sp_ref[1]
    h = (s + jnp.dot(a_ref[...], we_ref[...],
                     preferred_element_type=jnp.float32)) * dis16_ref[:, 0:1]
    h_ref[...] = h

    @pl.when(i == 0)
    def _():
        stats_ref[...] = jnp.zeros_like(stats_ref)

    stats_ref[0:1, :] += jnp.sum(h, axis=0, keepdims=True)
    stats_ref[1:2, :] += jnp.sum(h * h, axis=0, keepdims=True)


def _tc3a_call(sp, a, we, dis16):
    return pl.pallas_call(
        _tc3a_body,
        grid=(_TG,),
        in_specs=[
            pl.BlockSpec((_NC, _TBR, _H), lambda i: (0, i, 0)),
            pl.BlockSpec((_TBR, 16), lambda i: (i, 0)),
            pl.BlockSpec((_DE, _H), lambda i: (0, 0)),
            pl.BlockSpec((_TBR, 16), lambda i: (i, 0)),
        ],
        out_specs=[
            pl.BlockSpec((_TBR, _H), lambda i: (i, 0)),
            pl.BlockSpec((8, _H), lambda i: (0, 0)),
        ],
        out_shape=[jax.ShapeDtypeStruct((_N, _H), jnp.float32),
                   jax.ShapeDtypeStruct((8, _H), jnp.float32)],
    )(sp, a, we, dis16)


def _tc3b_body(h_ref, stats_ref, g_ref, bb_ref, out_ref):
    mean = stats_ref[0:1, :] * (1.0 / _N)
    var = stats_ref[1:2, :] * (1.0 / _N) - mean * mean
    sc = lax.rsqrt(var + 1e-5) * g_ref[...]
    out_ref[...] = (h_ref[...] - mean) * sc + bb_ref[...]


def _tc3b_call(h, stats, g, bb):
    return pl.pallas_call(
        _tc3b_body,
        grid=(_TG,),
        in_specs=[
            pl.BlockSpec((_TBR, _H), lambda i: (i, 0)),
            pl.BlockSpec((8, _H), lambda i: (0, 0)),
            pl.BlockSpec((1, _H), lambda i: (0, 0)),
            pl.BlockSpec((1, _H), lambda i: (0, 0)),
        ],
        out_specs=pl.BlockSpec((_TBR, _H), lambda i: (i, 0)),
        out_shape=jax.ShapeDtypeStruct((_N, _H), jnp.float32),
    )(h, stats, g, bb)


def kernel(x, edge_index, edge_attr, Wn0, We0, b0, g0, bb0,
           Wn1, We1, b1, g1, bb1):
    src2d = edge_index[0].reshape(_IDXR, 128)
    dst2d = edge_index[1].reshape(_IDXR, 128)
    ones = jnp.ones((128, 16), jnp.float32)
    z16 = jnp.zeros((_RPT, 16), jnp.float32)
    z128 = jnp.zeros((_CH, _H), jnp.float32)
    b0r = b0.reshape(1, _H)
    b1r = b1.reshape(1, _H)
    g0r = g0.reshape(1, _H)
    g1r = g1.reshape(1, _H)
    bb0r = bb0.reshape(1, _H)
    bb1r = bb1.reshape(1, _H)

    degp = _deg_call(dst2d, ones, z16)
    hq0, dis16 = _tc1_call(degp, x, Wn0, b0r)
    s0p, ap = _s0_call(hq0, dis16, edge_attr, src2d, dst2d, z128, z16)
    h0, a, stats0 = _tca_call(s0p, ap, We0, dis16)
    hq1 = _tc2b_call(h0, stats0, g0r, bb0r, Wn1, b1r, dis16)
    s1p = _s1_call(hq1, src2d, dst2d, z128)
    h1, stats1 = _tc3a_call(s1p, a, We1, dis16)
    return _tc3b_call(h1, stats1, g1r, bb1r)


# speedup vs baseline: 12.4518x; 12.4518x over previous
import jax
import jax.numpy as jnp
import numpy as np
from jax import lax
from jax.experimental import pallas as pl
from jax.experimental.pallas import tpu as pltpu
from jax.experimental.pallas import tpu_sc as plsc

_NC = 2
_NS = 16
_NW = _NC * _NS

_N = 10000
_E = 320000
_D = 128
_DE = 16
_H = 128

_IDXR = _E // 128
_NP = 10240
_RPT = _NP // _NS
_CH = 128

_BR4 = 4
_NB4 = _IDXR // _BR4
_IT4 = -(-_NB4 // _NW)
_BE4 = _BR4 * 128

_BR2 = 2
_NB2 = _IDXR // _BR2
_IT2 = -(-_NB2 // _NW)


def _sc_mesh():
    return plsc.VectorSubcoreMesh(
        core_axis_name="c", subcore_axis_name="s",
        num_cores=_NC, num_subcores=_NS)


def _deg_body(dst3d, ones_hbm, z16_hbm, degp, acc16, dst_v, ones_v, z16_v,
              sem):
    c = lax.axis_index("c")
    s = lax.axis_index("s")
    wid = c * _NS + s

    pltpu.sync_copy(z16_hbm, z16_v)
    pltpu.sync_copy(ones_hbm, ones_v)
    pltpu.sync_copy(z16_v, acc16.at[pl.ds(s * _RPT, _RPT)])
    plsc.subcore_barrier()

    def body(i, carry):
        bi = i * _NW + wid

        @pl.when(bi < _NB4)
        def _():
            pltpu.sync_copy(dst3d.at[bi], dst_v)
            for k in range(_BR4):
                pltpu.sync_copy(ones_v, acc16.at[dst_v.at[k]], add=True)

        return carry

    lax.fori_loop(0, _IT4, body, None)
    plsc.subcore_barrier()
    pltpu.sync_copy(acc16.at[pl.ds(s * _RPT, _RPT)], z16_v)
    pltpu.sync_copy(z16_v, degp.at[c, pl.ds(s * _RPT, _RPT)])


def _deg_call(dst3d, ones, z16):
    return pl.kernel(
        _deg_body,
        out_type=jax.ShapeDtypeStruct((_NC, _NP, 16), jnp.float32),
        mesh=_sc_mesh(),
        compiler_params=pltpu.CompilerParams(use_tc_tiling_on_sc=False),
        scratch_types=[
            pltpu.VMEM_SHARED((_NP, 16), jnp.float32),
            pltpu.VMEM((_BR4, 128), jnp.int32),
            pltpu.VMEM((128, 16), jnp.float32),
            pltpu.VMEM((_RPT, 16), jnp.float32),
            pltpu.SemaphoreType.DMA,
        ],
    )(dst3d, ones, z16)


def _sa_body(dis1d_hbm, ea_hbm, src3d, dst3d, z16_hbm, ap, acc16,
             src_v, dst_v, ea_v, dis_v, z16_v, seme):
    c = lax.axis_index("c")
    s = lax.axis_index("s")
    wid = c * _NS + s

    pltpu.sync_copy(z16_hbm, z16_v)
    pltpu.sync_copy(dis1d_hbm, dis_v)
    pltpu.sync_copy(z16_v, acc16.at[pl.ds(s * _RPT, _RPT)])
    plsc.subcore_barrier()

    def body(i, carry):
        bi = i * _NW + wid

        @pl.when(bi < _NB4)
        def _():
            pltpu.sync_copy(src3d.at[bi], src_v)
            pltpu.sync_copy(dst3d.at[bi], dst_v)
            ecp = pltpu.async_copy(ea_hbm.at[pl.ds(bi * _BE4, _BE4)], ea_v,
                                   seme)
            ecp.wait()

            def mbody(j, mc):
                iv = src_v[j // 8, pl.ds((j % 8) * 16, 16)]
                d16 = plsc.load_gather(dis_v, [iv])
                zero16 = lax.iota(jnp.int32, 16) * 0
                for t in range(16):
                    bt = jnp.take_along_axis(d16, zero16 + t, axis=0,
                                             mode="promise_in_bounds")
                    ea_v[j * 16 + t, :] = ea_v[j * 16 + t, :] * bt
                return mc

            lax.fori_loop(0, _BE4 // 16, mbody, None)
            for k in range(_BR4):
                pltpu.sync_copy(ea_v.at[pl.ds(k * 128, 128)],
                                acc16.at[dst_v.at[k]], add=True)

        return carry

    lax.fori_loop(0, _IT4, body, None)
    plsc.subcore_barrier()
    pltpu.sync_copy(acc16.at[pl.ds(s * _RPT, _RPT)], z16_v)
    pltpu.sync_copy(z16_v, ap.at[c, pl.ds(s * _RPT, _RPT)])


def _sa_call(dis1d, ea, src3d, dst3d, z16):
    return pl.kernel(
        _sa_body,
        out_type=jax.ShapeDtypeStruct((_NC, _NP, 16), jnp.float32),
        mesh=_sc_mesh(),
        compiler_params=pltpu.CompilerParams(needs_layout_passes=False,
                                             use_tc_tiling_on_sc=False),
        scratch_types=[
            pltpu.VMEM_SHARED((_NP, 16), jnp.float32),
            pltpu.VMEM((_BR4, 128), jnp.int32),
            pltpu.VMEM((_BR4, 128), jnp.int32),
            pltpu.VMEM((_BE4, 16), jnp.float32),
            pltpu.VMEM((_N,), jnp.float32),
            pltpu.VMEM((_RPT, 16), jnp.float32),
            pltpu.SemaphoreType.DMA,
        ],
    )(dis1d, ea, src3d, dst3d, z16)


def _s_body(hq_hbm, src3d, dst3d, z128_hbm, sp, acc, src_v, dst_v, rows_v,
            semg):
    c = lax.axis_index("c")
    s = lax.axis_index("s")
    wid = c * _NS + s

    pltpu.sync_copy(z128_hbm, rows_v.at[pl.ds(0, _CH)])
    for k in range(_RPT // _CH):
        pltpu.sync_copy(rows_v.at[pl.ds(0, _CH)],
                        acc.at[pl.ds(s * _RPT + k * _CH, _CH)])
    plsc.subcore_barrier()

    def body(i, carry):
        bi = i * _NW + wid

        @pl.when(bi < _NB2)
        def _():
            pltpu.sync_copy(src3d.at[bi], src_v)
            pltpu.sync_copy(dst3d.at[bi], dst_v)
            gcp = [pltpu.async_copy(hq_hbm.at[src_v.at[k]],
                                    rows_v.at[pl.ds(k * 128, 128)], semg)
                   for k in range(_BR2)]
            for k in range(_BR2):
                gcp[k].wait()
                pltpu.sync_copy(rows_v.at[pl.ds(k * 128, 128)],
                                acc.at[dst_v.at[k]], add=True)

        return carry

    lax.fori_loop(0, _IT2, body, None)
    plsc.subcore_barrier()
    for k in range(_RPT // _CH):
        pltpu.sync_copy(acc.at[pl.ds(s * _RPT + k * _CH, _CH)],
                        rows_v.at[pl.ds(0, _CH)])
        pltpu.sync_copy(rows_v.at[pl.ds(0, _CH)],
                        sp.at[c, pl.ds(s * _RPT + k * _CH, _CH)])


def _s_call(hq, src3d, dst3d, z128):
    return pl.kernel(
        _s_body,
        out_type=jax.ShapeDtypeStruct((_NC, _NP, _H), jnp.float32),
        mesh=_sc_mesh(),
        scratch_types=[
            pltpu.VMEM_SHARED((_NP, _H), jnp.float32),
            pltpu.VMEM((_BR2, 128), jnp.int32),
            pltpu.VMEM((_BR2, 128), jnp.int32),
            pltpu.VMEM((_BR2 * 128, _H), jnp.float32),
            pltpu.SemaphoreType.DMA,
        ],
    )(hq, src3d, dst3d, z128)


_TBR = 400
_TG = _N // _TBR


def _tc1_body(degp_ref, x_ref, wn_ref, b_ref, hq_ref, dis16_ref):
    deg16 = degp_ref[0] + degp_ref[1]
    dis16 = lax.rsqrt(deg16 + 1.0)
    hp = jnp.dot(x_ref[...], wn_ref[...],
                 preferred_element_type=jnp.float32) + b_ref[...]
    hq_ref[...] = hp * dis16[:, 0:1]
    dis16_ref[...] = dis16


def _tc1_call(degp, x, wn, b):
    return pl.pallas_call(
        _tc1_body,
        grid=(_TG,),
        in_specs=[
            pl.BlockSpec((_NC, _TBR, 16), lambda i: (0, i, 0)),
            pl.BlockSpec((_TBR, _D), lambda i: (i, 0)),
            pl.BlockSpec((_D, _H), lambda i: (0, 0)),
            pl.BlockSpec((1, _H), lambda i: (0, 0)),
        ],
        out_specs=[
            pl.BlockSpec((_TBR, _H), lambda i: (i, 0)),
            pl.BlockSpec((_TBR, 16), lambda i: (i, 0)),
        ],
        out_shape=[jax.ShapeDtypeStruct((_N, _H), jnp.float32),
                   jax.ShapeDtypeStruct((_N, 16), jnp.float32)],
    )(degp, x, wn, b)


def _tca_body(sp_ref, ap_ref, we_ref, dis16_ref, h_ref, a_ref, stats_ref):
    i = pl.program_id(0)
    a = ap_ref[0] + ap_ref[1]
    s = sp_ref[0] + sp_ref[1]
    h = (s + jnp.dot(a, we_ref[...], preferred_element_type=jnp.float32)
         ) * dis16_ref[:, 0:1]
    h_ref[...] = h
    a_ref[...] = a

    @pl.when(i == 0)
    def _():
        stats_ref[...] = jnp.zeros_like(stats_ref)

    stats_ref[0:1, :] += jnp.sum(h, axis=0, keepdims=True)
    stats_ref[1:2, :] += jnp.sum(h * h, axis=0, keepdims=True)


def _tca_call(sp, ap, we, dis16):
    return pl.pallas_call(
        _tca_body,
        grid=(_TG,),
        in_specs=[
            pl.BlockSpec((_NC, _TBR, _H), lambda i: (0, i, 0)),
            pl.BlockSpec((_NC, _TBR, 16), lambda i: (0, i, 0)),
            pl.BlockSpec((_DE, _H), lambda i: (0, 0)),
            pl.BlockSpec((_TBR, 16), lambda i: (i, 0)),
        ],
        out_specs=[
            pl.BlockSpec((_TBR, _H), lambda i: (i, 0)),
            pl.BlockSpec((_TBR, 16), lambda i: (i, 0)),
            pl.BlockSpec((8, _H), lambda i: (0, 0)),
        ],
        out_shape=[jax.ShapeDtypeStruct((_N, _H), jnp.float32),
                   jax.ShapeDtypeStruct((_N, 16), jnp.float32),
                   jax.ShapeDtypeStruct((8, _H), jnp.float32)],
    )(sp, ap, we, dis16)


def _tc2b_body(h_ref, stats_ref, g_ref, bb_ref, wn_ref, b_ref, dis16_ref,
               hq_ref):
    mean = stats_ref[0:1, :] * (1.0 / _N)
    var = stats_ref[1:2, :] * (1.0 / _N) - mean * mean
    sc = lax.rsqrt(var + 1e-5) * g_ref[...]
    bn = (h_ref[...] - mean) * sc + bb_ref[...]
    r = jnp.maximum(bn, 0.0)
    hq_ref[...] = (jnp.dot(r, wn_ref[...],
                           preferred_element_type=jnp.float32)
                   + b_ref[...]) * dis16_ref[:, 0:1]


def _tc2b_call(h, stats, g, bb, wn, b, dis16):
    return pl.pallas_call(
        _tc2b_body,
        grid=(_TG,),
        in_specs=[
            pl.BlockSpec((_TBR, _H), lambda i: (i, 0)),
            pl.BlockSpec((8, _H), lambda i: (0, 0)),
            pl.BlockSpec((1, _H), lambda i: (0, 0)),
            pl.BlockSpec((1, _H), lambda i: (0, 0)),
            pl.BlockSpec((_H, _H), lambda i: (0, 0)),
            pl.BlockSpec((1, _H), lambda i: (0, 0)),
            pl.BlockSpec((_TBR, 16), lambda i: (i, 0)),
        ],
        out_specs=pl.BlockSpec((_TBR, _H), lambda i: (i, 0)),
        out_shape=jax.ShapeDtypeStruct((_N, _H), jnp.float32),
    )(h, stats, g, bb, wn, b, dis16)


def _tc3a_body(sp_ref, a_ref, we_ref, dis16_ref, h_ref, stats_ref):
    i = pl.program_id(0)
    s = sp_ref[0] + sp_ref[1]
    h = (s + jnp.dot(a_ref[...], we_ref[...],
                     preferred_element_type=jnp.float32)) * dis16_ref[:, 0:1]
    h_ref[...] = h

    @pl.when(i == 0)
    def _():
        stats_ref[...] = jnp.zeros_like(stats_ref)

    stats_ref[0:1, :] += jnp.sum(h, axis=0, keepdims=True)
    stats_ref[1:2, :] += jnp.sum(h * h, axis=0, keepdims=True)


def _tc3a_call(sp, a, we, dis16):
    return pl.pallas_call(
        _tc3a_body,
        grid=(_TG,),
        in_specs=[
            pl.BlockSpec((_NC, _TBR, _H), lambda i: (0, i, 0)),
            pl.BlockSpec((_TBR, 16), lambda i: (i, 0)),
            pl.BlockSpec((_DE, _H), lambda i: (0, 0)),
            pl.BlockSpec((_TBR, 16), lambda i: (i, 0)),
        ],
        out_specs=[
            pl.BlockSpec((_TBR, _H), lambda i: (i, 0)),
            pl.BlockSpec((8, _H), lambda i: (0, 0)),
        ],
        out_shape=[jax.ShapeDtypeStruct((_N, _H), jnp.float32),
                   jax.ShapeDtypeStruct((8, _H), jnp.float32)],
    )(sp, a, we, dis16)


def _tc3b_body(h_ref, stats_ref, g_ref, bb_ref, out_ref):
    mean = stats_ref[0:1, :] * (1.0 / _N)
    var = stats_ref[1:2, :] * (1.0 / _N) - mean * mean
    sc = lax.rsqrt(var + 1e-5) * g_ref[...]
    out_ref[...] = (h_ref[...] - mean) * sc + bb_ref[...]


def _tc3b_call(h, stats, g, bb):
    return pl.pallas_call(
        _tc3b_body,
        grid=(_TG,),
        in_specs=[
            pl.BlockSpec((_TBR, _H), lambda i: (i, 0)),
            pl.BlockSpec((8, _H), lambda i: (0, 0)),
            pl.BlockSpec((1, _H), lambda i: (0, 0)),
            pl.BlockSpec((1, _H), lambda i: (0, 0)),
        ],
        out_specs=pl.BlockSpec((_TBR, _H), lambda i: (i, 0)),
        out_shape=jax.ShapeDtypeStruct((_N, _H), jnp.float32),
    )(h, stats, g, bb)


def kernel(x, edge_index, edge_attr, Wn0, We0, b0, g0, bb0,
           Wn1, We1, b1, g1, bb1):
    src4 = edge_index[0].reshape(_NB4, _BR4, 128)
    dst4 = edge_index[1].reshape(_NB4, _BR4, 128)
    src2 = edge_index[0].reshape(_NB2, _BR2, 128)
    dst2 = edge_index[1].reshape(_NB2, _BR2, 128)
    ones = jnp.ones((128, 16), jnp.float32)
    z16 = jnp.zeros((_RPT, 16), jnp.float32)
    z128 = jnp.zeros((_CH, _H), jnp.float32)
    b0r = b0.reshape(1, _H)
    b1r = b1.reshape(1, _H)
    g0r = g0.reshape(1, _H)
    g1r = g1.reshape(1, _H)
    bb0r = bb0.reshape(1, _H)
    bb1r = bb1.reshape(1, _H)

    degp = _deg_call(dst4, ones, z16)
    hq0, dis16 = _tc1_call(degp, x, Wn0, b0r)
    dis1d = dis16[:, 0]
    ap = _sa_call(dis1d, edge_attr, src4, dst4, z16)
    s0p = _s_call(hq0, src2, dst2, z128)
    h0, a, stats0 = _tca_call(s0p, ap, We0, dis16)
    hq1 = _tc2b_call(h0, stats0, g0r, bb0r, Wn1, b1r, dis16)
    s1p = _s_call(hq1, src2, dst2, z128)
    h1, stats1 = _tc3a_call(s1p, a, We1, dis16)
    return _tc3b_call(h1, stats1, g1r, bb1r)
